# baseline (speedup 1.0000x reference)
import functools

import jax
import jax.numpy as jnp
from jax import lax
from jax.experimental import pallas as pl
from jax.experimental.pallas import tpu as pltpu
from jax.experimental.pallas import tpu_sc as plsc


def _sc_gather_t(table_t, idx, n, d):
    ch = n
    nch = n // ch
    mesh = plsc.VectorSubcoreMesh(core_axis_name="c", subcore_axis_name="s")

    @functools.partial(
        pl.kernel,
        mesh=mesh,
        compiler_params=pltpu.CompilerParams(use_tc_tiling_on_sc=False),
        out_type=jax.ShapeDtypeStruct((d, n), jnp.float32),
        scratch_types=[
            pltpu.VMEM((n,), jnp.int32),
            pltpu.VMEM((n,), jnp.float32),
            pltpu.SemaphoreType.DMA,
        ],
    )
    def k(table_hbm, idx_hbm, out_hbm, idx_v, row_v, sem):
        wid = lax.axis_index("s") * mesh.num_cores + lax.axis_index("c")
        pltpu.sync_copy(idx_hbm, idx_v)
        src = table_hbm.at[wid]
        copies = [
            pltpu.async_copy(
                src.at[idx_v.at[pl.ds(j * ch, ch)]],
                row_v.at[pl.ds(j * ch, ch)],
                sem,
            )
            for j in range(nch)
        ]
        for c in copies:
            c.wait()
        pltpu.sync_copy(row_v, out_hbm.at[wid])

    return k(table_t, idx)


def _tc_body(mus_ref, rft_ref, ann_ref, conf_ref, out_ref):
    mu = mus_ref[...]
    rft = rft_ref[...]
    ann = ann_ref[...]
    conf = conf_ref[...]
    b = rft.shape[1]
    dn = (((1,), (0,)), ((), ()))
    p8 = lax.dot_general(jnp.exp(mu), jnp.exp(rft), dn,
                         preferred_element_type=jnp.float32)
    iota_d = lax.broadcasted_iota(jnp.int32, (32, b), 0)
    oht = (iota_d == ann).astype(jnp.float32)
    rpt = jnp.sum(rft * oht, axis=0, keepdims=True)
    mpt = lax.dot_general(mu, oht, dn,
                          preferred_element_type=jnp.float32)
    out_ref[...] = conf * (mpt + rpt - jnp.log(p8))


def _tc_compute(rft, mus, ann_row, conf_row, b):
    d, n = rft.shape
    c = mus.shape[0]
    grid = n // b
    return pl.pallas_call(
        _tc_body,
        grid=(grid,),
        in_specs=[
            pl.BlockSpec((c, d), lambda i: (0, 0)),
            pl.BlockSpec((d, b), lambda i: (0, i)),
            pl.BlockSpec((1, b), lambda i: (0, i)),
            pl.BlockSpec((1, b), lambda i: (0, i)),
        ],
        out_specs=pl.BlockSpec((c, b), lambda i: (0, i)),
        out_shape=jax.ShapeDtypeStruct((c, n), jnp.float32),
    )(mus, rft, ann_row, conf_row)


def kernel(mus, random_effects, annotators, annotations, confidences):
    n = annotators.shape[0]
    d = random_effects.shape[1]
    rft = _sc_gather_t(random_effects.T, annotators, n, d)
    ann_row = annotations.reshape(1, n)
    conf_row = confidences.reshape(1, n)
    out_t = _tc_compute(rft, mus, ann_row, conf_row, b=16384)
    return out_t.T


# speedup vs baseline: 1.0012x; 1.0012x over previous
import functools

import jax
import jax.numpy as jnp
from jax import lax
from jax.experimental import pallas as pl
from jax.experimental.pallas import tpu as pltpu
from jax.experimental.pallas import tpu_sc as plsc


def _sc_gather_t(table_t, idx, n, d):
    ch = n
    nch = n // ch
    mesh = plsc.VectorSubcoreMesh(core_axis_name="c", subcore_axis_name="s")

    @functools.partial(
        pl.kernel,
        mesh=mesh,
        compiler_params=pltpu.CompilerParams(use_tc_tiling_on_sc=False),
        out_type=jax.ShapeDtypeStruct((d, n), jnp.float32),
        scratch_types=[
            pltpu.VMEM((n,), jnp.int32),
            pltpu.VMEM((n,), jnp.float32),
            pltpu.SemaphoreType.DMA,
        ],
    )
    def k(table_hbm, idx_hbm, out_hbm, idx_v, row_v, sem):
        wid = lax.axis_index("s") * mesh.num_cores + lax.axis_index("c")
        pltpu.sync_copy(idx_hbm, idx_v)
        src = table_hbm.at[wid]
        copies = [
            pltpu.async_copy(
                src.at[idx_v.at[pl.ds(j * ch, ch)]],
                row_v.at[pl.ds(j * ch, ch)],
                sem,
            )
            for j in range(nch)
        ]
        for c in copies:
            c.wait()
        pltpu.sync_copy(row_v, out_hbm.at[wid])

    return k(table_t, idx)


def _tc_body(mus_ref, rft_ref, ann_ref, conf_ref, out_ref):
    mu = mus_ref[...]
    rft = rft_ref[...]
    ann = ann_ref[...]
    conf = conf_ref[...]
    b = rft.shape[1]
    dn = (((1,), (0,)), ((), ()))
    p8 = lax.dot_general(jnp.exp(mu), jnp.exp(rft), dn,
                         preferred_element_type=jnp.float32)
    iota_d = lax.broadcasted_iota(jnp.int32, (32, b), 0)
    oht = (iota_d == ann).astype(jnp.float32)
    rpt = jnp.sum(rft * oht, axis=0, keepdims=True)
    mpt = lax.dot_general(mu, oht, dn,
                          preferred_element_type=jnp.float32)
    out_ref[...] = conf * (mpt + rpt - jnp.log(p8))


def _tc_compute(rft, mus, ann_row, conf_row, b):
    d, n = rft.shape
    c = mus.shape[0]
    grid = n // b
    return pl.pallas_call(
        _tc_body,
        grid=(grid,),
        in_specs=[
            pl.BlockSpec((c, d), lambda i: (0, 0)),
            pl.BlockSpec((d, b), lambda i: (0, i)),
            pl.BlockSpec((1, b), lambda i: (0, i)),
            pl.BlockSpec((1, b), lambda i: (0, i)),
        ],
        out_specs=pl.BlockSpec((c, b), lambda i: (0, i)),
        out_shape=jax.ShapeDtypeStruct((c, n), jnp.float32),
    )(mus, rft, ann_row, conf_row)


def kernel(mus, random_effects, annotators, annotations, confidences):
    n = annotators.shape[0]
    d = random_effects.shape[1]
    rft = _sc_gather_t(random_effects.T, annotators, n, d)
    ann_row = annotations.reshape(1, n)
    conf_row = confidences.reshape(1, n)
    out_t = _tc_compute(rft, mus, ann_row, conf_row, b=8192)
    return out_t.T
